# baseline (speedup 1.0000x reference)
import functools

import jax
import jax.numpy as jnp
from jax import lax
from jax.experimental import pallas as pl
from jax.experimental.pallas import tpu as pltpu
from jax.experimental.pallas import tpu_sc as plsc

N = 10000
E = 320000
R = 8
D = 128

NP = 10240
EP = 327680
NW = 32
EPW = EP // NW
CH = 512
NCHUNK = EPW // CH
SUP = 16
NSUP = EPW // (128 * SUP)
ROWS_PER_TILE = NP // 16
CNT_SZ = R * NP
CNT_PER_TILE = CNT_SZ // 16


def _zero16(ref, i):
    ref[pl.ds(i * 16, 16)] = jnp.zeros((16,), jnp.float32)


def _prep_body(typ2d, dst2d, src2d, fidx_out, w_out,
               cnt_sh, zbuf, tbuf, dbuf, sbuf, fbuf, wbuf, ones, invtab, dsem):
    c = lax.axis_index("c")
    s = lax.axis_index("s")
    wid = s * 2 + c

    @pl.loop(0, CNT_PER_TILE // 16)
    def _(i):
        _zero16(zbuf, i)
    for i in range(128 // 16):
        ones[pl.ds(i * 16, 16)] = jnp.ones((16,), jnp.float32)
    pltpu.sync_copy(zbuf, cnt_sh.at[pl.ds(s * CNT_PER_TILE, CNT_PER_TILE)])
    plsc.subcore_barrier()

    rows_per_sub = (EP // 16) // 128
    @pl.loop(0, rows_per_sub // 8)
    def _(ch):
        rb = s * rows_per_sub + ch * 8
        ld = [pltpu.async_copy(typ2d.at[pl.ds(rb, 8)], tbuf, dsem),
              pltpu.async_copy(dst2d.at[pl.ds(rb, 8)], dbuf, dsem)]
        for d_ in ld:
            d_.wait()
        for j in range(8):
            for t in range(8):
                sl = pl.ds(t * 16, 16)
                fbuf[j, sl] = tbuf[j, sl] * NP + dbuf[j, sl]
        descs = [pltpu.async_copy(ones, cnt_sh.at[fbuf.at[j]], dsem, add=True)
                 for j in range(8)]
        for d_ in descs:
            d_.wait()
    plsc.subcore_barrier()

    pltpu.sync_copy(cnt_sh.at[pl.ds(s * CNT_PER_TILE, CNT_PER_TILE)], zbuf)
    @pl.loop(0, CNT_PER_TILE // 16)
    def _(i):
        sl = pl.ds(i * 16, 16)
        zbuf[sl] = 1.0 / jnp.maximum(zbuf[sl], 1.0)
    pltpu.sync_copy(zbuf, cnt_sh.at[pl.ds(s * CNT_PER_TILE, CNT_PER_TILE)])
    plsc.subcore_barrier()

    pltpu.sync_copy(cnt_sh, invtab)

    @pl.loop(0, (EPW // 128) // 8)
    def _(ch):
        rb = wid * (EPW // 128) + ch * 8
        ld = [pltpu.async_copy(typ2d.at[pl.ds(rb, 8)], tbuf, dsem),
              pltpu.async_copy(dst2d.at[pl.ds(rb, 8)], dbuf, dsem),
              pltpu.async_copy(src2d.at[pl.ds(rb, 8)], sbuf, dsem)]
        for d_ in ld:
            d_.wait()
        for j in range(8):
            for t in range(8):
                sl = pl.ds(t * 16, 16)
                tv = tbuf[j, sl]
                fbuf[j, sl] = tv * NP + sbuf[j, sl]
                cv = tv * NP + dbuf[j, sl]
                wbuf[pl.ds(j * 128 + t * 16, 16)] = plsc.load_gather(invtab, [cv])
        pltpu.sync_copy(fbuf, fidx_out.at[pl.ds(rb, 8)])
        pltpu.sync_copy(wbuf, w_out.at[pl.ds(wid * EPW + ch * 1024, 1024)])


@functools.cache
def _get_prep():
    mesh = plsc.VectorSubcoreMesh(core_axis_name="c", subcore_axis_name="s")
    return pl.kernel(
        _prep_body,
        out_type=(
            jax.ShapeDtypeStruct((EP // 128, 128), jnp.int32),
            jax.ShapeDtypeStruct((EP,), jnp.float32),
        ),
        mesh=mesh,
        scratch_types=(
            pltpu.VMEM_SHARED((CNT_SZ,), jnp.float32),
            pltpu.VMEM((CNT_PER_TILE,), jnp.float32),
            pltpu.VMEM((8, 128), jnp.int32),
            pltpu.VMEM((8, 128), jnp.int32),
            pltpu.VMEM((8, 128), jnp.int32),
            pltpu.VMEM((8, 128), jnp.int32),
            pltpu.VMEM((1024,), jnp.float32),
            pltpu.VMEM((128,), jnp.float32),
            pltpu.VMEM((CNT_SZ,), jnp.float32),
            pltpu.SemaphoreType.DMA,
        ),
        compiler_params=pltpu.CompilerParams(needs_layout_passes=False),
        name="rgcn_prep",
    )


def _agg_body(hflat, fidx2d, dst2d, w_hbm, out, acc_sh, fibuf, dbuf, wbuf, rows,
              zbuf, gs0, gs1, ss0, ss1):
    c = lax.axis_index("c")
    s = lax.axis_index("s")
    wid = s * 2 + c

    for i in range(16):
        for t in range(8):
            zbuf[i, pl.ds(t * 16, 16)] = jnp.zeros((16,), jnp.float32)
    for g in range(ROWS_PER_TILE // 16 // 8):
        zd = [pltpu.async_copy(
                  zbuf, acc_sh.at[pl.ds(s * ROWS_PER_TILE + (g * 8 + k) * 16, 16)],
                  gs0)
              for k in range(8)]
        for d_ in zd:
            d_.wait()
    plsc.subcore_barrier()

    gsems = (gs0, gs1)
    ssems = (ss0, ss1)

    @pl.loop(0, NSUP)
    def _(sp):
        rb = wid * (EPW // 128) + sp * SUP
        ld = [pltpu.async_copy(fidx2d.at[pl.ds(rb, SUP)], fibuf, gs1),
              pltpu.async_copy(dst2d.at[pl.ds(rb, SUP)], dbuf, gs1),
              pltpu.async_copy(
                  w_hbm.at[pl.ds(wid * EPW + sp * (SUP * 128), SUP * 128)],
                  wbuf, gs1)]
        for d_ in ld:
            d_.wait()
        gd = [None, None]
        sd = [None, None]
        gd[0] = pltpu.async_copy(hflat.at[fibuf.at[0]], rows.at[pl.ds(0, 128)],
                                 gsems[0])
        for j in range(SUP):
            b = j % 2
            nb = (j + 1) % 2
            if j + 1 < SUP:
                if sd[nb] is not None:
                    sd[nb].wait()
                gd[nb] = pltpu.async_copy(hflat.at[fibuf.at[j + 1]],
                                          rows.at[pl.ds(nb * 128, 128)],
                                          gsems[nb])
            gd[b].wait()

            @plsc.parallel_loop(0, 128, unroll=16)
            def _(e, _j=j, _b=b):
                wv = plsc.load_gather(
                    wbuf, [jnp.broadcast_to(_j * 128 + e, (16,))])
                for t in range(8):
                    sl = pl.ds(t * 16, 16)
                    rows[_b * 128 + e, sl] = rows[_b * 128 + e, sl] * wv

            sd[b] = pltpu.async_copy(rows.at[pl.ds(b * 128, 128)],
                                     acc_sh.at[dbuf.at[j]], ssems[b], add=True)
        sd[0].wait()
        sd[1].wait()
    plsc.subcore_barrier()
    pltpu.sync_copy(acc_sh.at[pl.ds(s * ROWS_PER_TILE, ROWS_PER_TILE)],
                    out.at[c, pl.ds(s * ROWS_PER_TILE, ROWS_PER_TILE)])


@functools.cache
def _get_agg():
    mesh = plsc.VectorSubcoreMesh(core_axis_name="c", subcore_axis_name="s")
    return pl.kernel(
        _agg_body,
        out_type=jax.ShapeDtypeStruct((2, NP, D), jnp.float32),
        mesh=mesh,
        scratch_types=(
            pltpu.VMEM_SHARED((NP, D), jnp.float32),
            pltpu.VMEM((SUP, 128), jnp.int32),
            pltpu.VMEM((SUP, 128), jnp.int32),
            pltpu.VMEM((SUP * 128,), jnp.float32),
            pltpu.VMEM((2 * 128, D), jnp.float32),
            pltpu.VMEM((16, D), jnp.float32),
            pltpu.SemaphoreType.DMA,
            pltpu.SemaphoreType.DMA,
            pltpu.SemaphoreType.DMA,
            pltpu.SemaphoreType.DMA,
        ),
        compiler_params=pltpu.CompilerParams(needs_layout_passes=False),
        name="rgcn_agg",
    )


BM = 512
NB = NP // BM


def _mm_first_body(x_ref, w_ref, b_ref, out_ref):
    x = x_ref[...]
    for r in range(R + 1):
        h = jnp.dot(x, w_ref[r], preferred_element_type=jnp.float32)
        if r == R:
            h = h + b_ref[...]
        out_ref[r] = h


def _mm_fused_body(hprev_ref, p_ref, w_ref, b_ref, out_ref):
    x = jax.nn.relu(hprev_ref[0] + p_ref[0] + p_ref[1])
    for r in range(R + 1):
        h = jnp.dot(x, w_ref[r], preferred_element_type=jnp.float32)
        if r == R:
            h = h + b_ref[...]
        out_ref[r] = h


_mm_first = pl.pallas_call(
    _mm_first_body,
    grid=(NB,),
    in_specs=[
        pl.BlockSpec((BM, D), lambda i: (i, 0)),
        pl.BlockSpec((R + 1, D, D), lambda i: (0, 0, 0)),
        pl.BlockSpec((1, D), lambda i: (0, 0)),
    ],
    out_specs=pl.BlockSpec((R + 1, BM, D), lambda i: (0, i, 0)),
    out_shape=jax.ShapeDtypeStruct((R + 1, NP, D), jnp.float32),
)

_mm_fused = pl.pallas_call(
    _mm_fused_body,
    grid=(NB,),
    in_specs=[
        pl.BlockSpec((1, BM, D), lambda i: (R, i, 0)),
        pl.BlockSpec((2, BM, D), lambda i: (0, i, 0)),
        pl.BlockSpec((R + 1, D, D), lambda i: (0, 0, 0)),
        pl.BlockSpec((1, D), lambda i: (0, 0)),
    ],
    out_specs=pl.BlockSpec((R + 1, BM, D), lambda i: (0, i, 0)),
    out_shape=jax.ShapeDtypeStruct((R + 1, NP, D), jnp.float32),
)

FBM = 1000


def _final_body(h_ref, p_ref, out_ref):
    out_ref[...] = h_ref[0] + p_ref[0] + p_ref[1]


_final = pl.pallas_call(
    _final_body,
    grid=(N // FBM,),
    in_specs=[
        pl.BlockSpec((1, FBM, D), lambda i: (R, i, 0)),
        pl.BlockSpec((2, FBM, D), lambda i: (0, i, 0)),
    ],
    out_specs=pl.BlockSpec((FBM, D), lambda i: (i, 0)),
    out_shape=jax.ShapeDtypeStruct((N, D), jnp.float32),
)


def kernel(x, edge_index, edge_type, W1, root1, b1, W2, root2, b2, W3, root3, b3):
    f32 = jnp.float32
    xp = jnp.zeros((NP, D), f32).at[:N].set(x.astype(f32))

    src = edge_index[0].astype(jnp.int32)
    dst = edge_index[1].astype(jnp.int32)
    typ = edge_type.astype(jnp.int32)

    npad = EP - E
    i = jnp.arange(npad, dtype=jnp.int32)
    src_p = jnp.concatenate([src, i % N])
    dst_p = jnp.concatenate([dst, N + (i % 128)])
    typ_p = jnp.concatenate([typ, i % R])

    src2d = src_p.reshape(EP // 128, 128)
    dst2d = dst_p.reshape(EP // 128, 128)
    typ2d = typ_p.reshape(EP // 128, 128)

    fidx2d, w = _get_prep()(typ2d, dst2d, src2d)

    def layer(Wl, rootl, bl, hprev, pprev):
        wall = jnp.concatenate([Wl.astype(f32), rootl.astype(f32)[None]], axis=0)
        b2d = bl.astype(f32).reshape(1, D)
        if hprev is None:
            h = _mm_first(xp, wall, b2d)
        else:
            h = _mm_fused(hprev, pprev, wall, b2d)
        p = _get_agg()(h.reshape((R + 1) * NP, D), fidx2d, dst2d, w)
        return h, p

    h1, p1 = layer(W1, root1, b1, None, None)
    h2, p2 = layer(W2, root2, b2, h1, p1)
    h3, p3 = layer(W3, root3, b3, h2, p2)
    return _final(h3, p3)


# speedup vs baseline: 1.0486x; 1.0486x over previous
import functools

import jax
import jax.numpy as jnp
from jax import lax
from jax.experimental import pallas as pl
from jax.experimental.pallas import tpu as pltpu
from jax.experimental.pallas import tpu_sc as plsc

N = 10000
E = 320000
R = 8
D = 128

NP = 10240
EP = 327680
NW = 32
EPW = EP // NW
CH = 512
NCHUNK = EPW // CH
SUP = 16
NSUP = EPW // (128 * SUP)
ROWS_PER_TILE = NP // 16
CNT_SZ = R * NP
CNT_PER_TILE = CNT_SZ // 16


def _zero16(ref, i):
    ref[pl.ds(i * 16, 16)] = jnp.zeros((16,), jnp.float32)


def _prep_body(typ2d, dst2d, src2d, fidx_out, w_out,
               cnt_sh, zbuf, tbuf, dbuf, sbuf, fbuf, wbuf, ones, invtab, dsem):
    c = lax.axis_index("c")
    s = lax.axis_index("s")
    wid = s * 2 + c

    @pl.loop(0, CNT_PER_TILE // 16)
    def _(i):
        _zero16(zbuf, i)
    for i in range(128 // 16):
        ones[pl.ds(i * 16, 16)] = jnp.ones((16,), jnp.float32)
    pltpu.sync_copy(zbuf, cnt_sh.at[pl.ds(s * CNT_PER_TILE, CNT_PER_TILE)])
    plsc.subcore_barrier()

    rows_per_sub = (EP // 16) // 128
    @pl.loop(0, rows_per_sub // 16)
    def _(ch):
        rb = s * rows_per_sub + ch * 16
        ld = [pltpu.async_copy(typ2d.at[pl.ds(rb, 16)], tbuf, dsem),
              pltpu.async_copy(dst2d.at[pl.ds(rb, 16)], dbuf, dsem)]
        for d_ in ld:
            d_.wait()
        for j in range(16):
            for t in range(8):
                sl = pl.ds(t * 16, 16)
                fbuf[j, sl] = tbuf[j, sl] * NP + dbuf[j, sl]
        descs = [pltpu.async_copy(ones, cnt_sh.at[fbuf.at[j]], dsem, add=True)
                 for j in range(16)]
        for d_ in descs:
            d_.wait()
    plsc.subcore_barrier()

    pltpu.sync_copy(cnt_sh.at[pl.ds(s * CNT_PER_TILE, CNT_PER_TILE)], zbuf)
    @pl.loop(0, CNT_PER_TILE // 16)
    def _(i):
        sl = pl.ds(i * 16, 16)
        zbuf[sl] = 1.0 / jnp.maximum(zbuf[sl], 1.0)
    pltpu.sync_copy(zbuf, cnt_sh.at[pl.ds(s * CNT_PER_TILE, CNT_PER_TILE)])
    plsc.subcore_barrier()

    pltpu.sync_copy(cnt_sh, invtab)

    @pl.loop(0, (EPW // 128) // 16)
    def _(ch):
        rb = wid * (EPW // 128) + ch * 16
        ld = [pltpu.async_copy(typ2d.at[pl.ds(rb, 16)], tbuf, dsem),
              pltpu.async_copy(dst2d.at[pl.ds(rb, 16)], dbuf, dsem),
              pltpu.async_copy(src2d.at[pl.ds(rb, 16)], sbuf, dsem)]
        for d_ in ld:
            d_.wait()
        for j in range(16):
            for t in range(8):
                sl = pl.ds(t * 16, 16)
                tv = tbuf[j, sl]
                fbuf[j, sl] = tv * NP + sbuf[j, sl]
                cv = tv * NP + dbuf[j, sl]
                wbuf[pl.ds(j * 128 + t * 16, 16)] = plsc.load_gather(invtab, [cv])
        pltpu.sync_copy(fbuf, fidx_out.at[pl.ds(rb, 16)])
        pltpu.sync_copy(wbuf, w_out.at[pl.ds(wid * EPW + ch * 2048, 2048)])


@functools.cache
def _get_prep():
    mesh = plsc.VectorSubcoreMesh(core_axis_name="c", subcore_axis_name="s")
    return pl.kernel(
        _prep_body,
        out_type=(
            jax.ShapeDtypeStruct((EP // 128, 128), jnp.int32),
            jax.ShapeDtypeStruct((EP,), jnp.float32),
        ),
        mesh=mesh,
        scratch_types=(
            pltpu.VMEM_SHARED((CNT_SZ,), jnp.float32),
            pltpu.VMEM((CNT_PER_TILE,), jnp.float32),
            pltpu.VMEM((16, 128), jnp.int32),
            pltpu.VMEM((16, 128), jnp.int32),
            pltpu.VMEM((16, 128), jnp.int32),
            pltpu.VMEM((16, 128), jnp.int32),
            pltpu.VMEM((2048,), jnp.float32),
            pltpu.VMEM((128,), jnp.float32),
            pltpu.VMEM((CNT_SZ,), jnp.float32),
            pltpu.SemaphoreType.DMA,
        ),
        compiler_params=pltpu.CompilerParams(needs_layout_passes=False),
        name="rgcn_prep",
    )


def _agg_body(hflat, fidx2d, dst2d, w_hbm, out, acc_sh, fibuf, dbuf, wbuf, rows,
              zbuf, gs0, gs1, ss0, ss1):
    c = lax.axis_index("c")
    s = lax.axis_index("s")
    wid = s * 2 + c

    for i in range(16):
        for t in range(8):
            zbuf[i, pl.ds(t * 16, 16)] = jnp.zeros((16,), jnp.float32)
    for g in range(ROWS_PER_TILE // 16 // 8):
        zd = [pltpu.async_copy(
                  zbuf, acc_sh.at[pl.ds(s * ROWS_PER_TILE + (g * 8 + k) * 16, 16)],
                  gs0)
              for k in range(8)]
        for d_ in zd:
            d_.wait()
    plsc.subcore_barrier()

    gsems = (gs0, gs1)
    ssems = (ss0, ss1)

    @pl.loop(0, NSUP)
    def _(sp):
        rb = wid * (EPW // 128) + sp * SUP
        ld = [pltpu.async_copy(fidx2d.at[pl.ds(rb, SUP)], fibuf, gs1),
              pltpu.async_copy(dst2d.at[pl.ds(rb, SUP)], dbuf, gs1),
              pltpu.async_copy(
                  w_hbm.at[pl.ds(wid * EPW + sp * (SUP * 128), SUP * 128)],
                  wbuf, gs1)]
        for d_ in ld:
            d_.wait()
        gd = [None, None]
        sd = [None, None]
        gd[0] = pltpu.async_copy(hflat.at[fibuf.at[0]], rows.at[pl.ds(0, 128)],
                                 gsems[0])
        for j in range(SUP):
            b = j % 2
            nb = (j + 1) % 2
            if j + 1 < SUP:
                if sd[nb] is not None:
                    sd[nb].wait()
                gd[nb] = pltpu.async_copy(hflat.at[fibuf.at[j + 1]],
                                          rows.at[pl.ds(nb * 128, 128)],
                                          gsems[nb])
            gd[b].wait()

            @plsc.parallel_loop(0, 128, unroll=8)
            def _(e, _j=j, _b=b):
                wv = plsc.load_gather(
                    wbuf, [jnp.broadcast_to(_j * 128 + e, (16,))])
                for t in range(8):
                    sl = pl.ds(t * 16, 16)
                    rows[_b * 128 + e, sl] = rows[_b * 128 + e, sl] * wv

            sd[b] = pltpu.async_copy(rows.at[pl.ds(b * 128, 128)],
                                     acc_sh.at[dbuf.at[j]], ssems[b], add=True)
        sd[0].wait()
        sd[1].wait()
    plsc.subcore_barrier()
    pltpu.sync_copy(acc_sh.at[pl.ds(s * ROWS_PER_TILE, ROWS_PER_TILE)],
                    out.at[c, pl.ds(s * ROWS_PER_TILE, ROWS_PER_TILE)])


@functools.cache
def _get_agg():
    mesh = plsc.VectorSubcoreMesh(core_axis_name="c", subcore_axis_name="s")
    return pl.kernel(
        _agg_body,
        out_type=jax.ShapeDtypeStruct((2, NP, D), jnp.float32),
        mesh=mesh,
        scratch_types=(
            pltpu.VMEM_SHARED((NP, D), jnp.float32),
            pltpu.VMEM((SUP, 128), jnp.int32),
            pltpu.VMEM((SUP, 128), jnp.int32),
            pltpu.VMEM((SUP * 128,), jnp.float32),
            pltpu.VMEM((2 * 128, D), jnp.float32),
            pltpu.VMEM((16, D), jnp.float32),
            pltpu.SemaphoreType.DMA,
            pltpu.SemaphoreType.DMA,
            pltpu.SemaphoreType.DMA,
            pltpu.SemaphoreType.DMA,
        ),
        compiler_params=pltpu.CompilerParams(needs_layout_passes=False),
        name="rgcn_agg",
    )


BM = 512
NB = NP // BM


def _mm_first_body(x_ref, w_ref, b_ref, out_ref):
    x = x_ref[...]
    for r in range(R + 1):
        h = jnp.dot(x, w_ref[r], preferred_element_type=jnp.float32)
        if r == R:
            h = h + b_ref[...]
        out_ref[r] = h


def _mm_fused_body(hprev_ref, p_ref, w_ref, b_ref, out_ref):
    x = jax.nn.relu(hprev_ref[0] + p_ref[0] + p_ref[1])
    for r in range(R + 1):
        h = jnp.dot(x, w_ref[r], preferred_element_type=jnp.float32)
        if r == R:
            h = h + b_ref[...]
        out_ref[r] = h


_mm_first = pl.pallas_call(
    _mm_first_body,
    grid=(NB,),
    in_specs=[
        pl.BlockSpec((BM, D), lambda i: (i, 0)),
        pl.BlockSpec((R + 1, D, D), lambda i: (0, 0, 0)),
        pl.BlockSpec((1, D), lambda i: (0, 0)),
    ],
    out_specs=pl.BlockSpec((R + 1, BM, D), lambda i: (0, i, 0)),
    out_shape=jax.ShapeDtypeStruct((R + 1, NP, D), jnp.float32),
)

_mm_fused = pl.pallas_call(
    _mm_fused_body,
    grid=(NB,),
    in_specs=[
        pl.BlockSpec((1, BM, D), lambda i: (R, i, 0)),
        pl.BlockSpec((2, BM, D), lambda i: (0, i, 0)),
        pl.BlockSpec((R + 1, D, D), lambda i: (0, 0, 0)),
        pl.BlockSpec((1, D), lambda i: (0, 0)),
    ],
    out_specs=pl.BlockSpec((R + 1, BM, D), lambda i: (0, i, 0)),
    out_shape=jax.ShapeDtypeStruct((R + 1, NP, D), jnp.float32),
)

FBM = 1000


def _final_body(h_ref, p_ref, out_ref):
    out_ref[...] = h_ref[0] + p_ref[0] + p_ref[1]


_final = pl.pallas_call(
    _final_body,
    grid=(N // FBM,),
    in_specs=[
        pl.BlockSpec((1, FBM, D), lambda i: (R, i, 0)),
        pl.BlockSpec((2, FBM, D), lambda i: (0, i, 0)),
    ],
    out_specs=pl.BlockSpec((FBM, D), lambda i: (i, 0)),
    out_shape=jax.ShapeDtypeStruct((N, D), jnp.float32),
)


def kernel(x, edge_index, edge_type, W1, root1, b1, W2, root2, b2, W3, root3, b3):
    f32 = jnp.float32
    xp = jnp.zeros((NP, D), f32).at[:N].set(x.astype(f32))

    src = edge_index[0].astype(jnp.int32)
    dst = edge_index[1].astype(jnp.int32)
    typ = edge_type.astype(jnp.int32)

    npad = EP - E
    i = jnp.arange(npad, dtype=jnp.int32)
    src_p = jnp.concatenate([src, i % N])
    dst_p = jnp.concatenate([dst, N + (i % 128)])
    typ_p = jnp.concatenate([typ, i % R])

    src2d = src_p.reshape(EP // 128, 128)
    dst2d = dst_p.reshape(EP // 128, 128)
    typ2d = typ_p.reshape(EP // 128, 128)

    fidx2d, w = _get_prep()(typ2d, dst2d, src2d)

    def layer(Wl, rootl, bl, hprev, pprev):
        wall = jnp.concatenate([Wl.astype(f32), rootl.astype(f32)[None]], axis=0)
        b2d = bl.astype(f32).reshape(1, D)
        if hprev is None:
            h = _mm_first(xp, wall, b2d)
        else:
            h = _mm_fused(hprev, pprev, wall, b2d)
        p = _get_agg()(h.reshape((R + 1) * NP, D), fidx2d, dst2d, w)
        return h, p

    h1, p1 = layer(W1, root1, b1, None, None)
    h2, p2 = layer(W2, root2, b2, h1, p1)
    h3, p3 = layer(W3, root3, b3, h2, p2)
    return _final(h3, p3)
